# baseline (speedup 1.0000x reference)
import jax
import jax.numpy as jnp
from jax.experimental import pallas as pl
from jax.experimental.pallas import tpu as pltpu

N = 10000
D = 128
BM = 80
NI = N // BM


def _gcn_kernel(a_ref, f_ref, w_ref, b_ref, out_ref, h_scr, norm_scr):
    phase = pl.program_id(0)
    i = pl.program_id(1)

    @pl.when(phase == 0)
    def _deg_h():
        deg = jnp.sum(a_ref[...], axis=1, keepdims=True)
        norm = jnp.where(deg > 0.0, jax.lax.rsqrt(deg), 0.0)
        norm_scr[pl.ds(i * BM, BM), :] = norm
        fw = jax.lax.dot_general(
            f_ref[...], w_ref[...],
            dimension_numbers=(((1,), (1,)), ((), ())),
            preferred_element_type=jnp.float32,
        )
        h_scr[pl.ds(i * BM, BM), :] = fw * norm

    @pl.when(phase == 1)
    def _spmm():
        acc = jnp.dot(a_ref[...], h_scr[...],
                      preferred_element_type=jnp.float32)
        out_ref[...] = acc * norm_scr[pl.ds(i * BM, BM), :] + b_ref[...]


def kernel(Adjacency, Features, W, b):
    assert Adjacency.shape == (N, N)
    assert Features.shape == (N, D)

    out = pl.pallas_call(
        _gcn_kernel,
        grid=(2, NI),
        in_specs=[
            pl.BlockSpec((BM, N), lambda p, i: (i, 0)),
            pl.BlockSpec((BM, D), lambda p, i: (i, 0)),
            pl.BlockSpec((D, D), lambda p, i: (0, 0)),
            pl.BlockSpec((1, D), lambda p, i: (0, 0)),
        ],
        out_specs=pl.BlockSpec((BM, D), lambda p, i: (p * i, 0)),
        out_shape=jax.ShapeDtypeStruct((N, D), jnp.float32),
        scratch_shapes=[
            pltpu.VMEM((N, D), jnp.float32),
            pltpu.VMEM((N, 1), jnp.float32),
        ],
        compiler_params=pltpu.CompilerParams(
            dimension_semantics=("arbitrary", "arbitrary")),
    )(Adjacency, Features, W, b.reshape(1, D))
    return out


# speedup vs baseline: 1.3353x; 1.3353x over previous
import jax
import jax.numpy as jnp
from jax.experimental import pallas as pl
from jax.experimental.pallas import tpu as pltpu

N = 10000
D = 128
BM = 200
NI = N // BM
CI = 5


def _gcn_kernel(a_ref, f_ref, w_ref, b_ref, out_ref,
                h_scr, hb_scr, norm_scr, cache_scr):
    phase = pl.program_id(0)
    i = pl.program_id(1)

    @pl.when(phase == 0)
    def _deg_h():
        a = a_ref[...]
        deg = jnp.sum(a, axis=1, keepdims=True)
        norm = jnp.where(deg > 0.0, jax.lax.rsqrt(deg), 0.0)
        norm_scr[pl.ds(i * BM, BM), :] = norm
        fw = jax.lax.dot_general(
            f_ref[...], w_ref[...],
            dimension_numbers=(((1,), (1,)), ((), ())),
            preferred_element_type=jnp.float32,
        )
        h = fw * norm
        h_scr[pl.ds(i * BM, BM), :] = h
        hb_scr[pl.ds(i * BM, BM), :] = h.astype(jnp.bfloat16)

        @pl.when(i >= NI - CI)
        def _stash():
            cache_scr[pl.ds((i - (NI - CI)) * BM, BM), :] = (
                a.astype(jnp.bfloat16))

    @pl.when(phase == 1)
    def _spmm():
        norm = norm_scr[pl.ds(i * BM, BM), :]

        @pl.when(i < NI - CI)
        def _from_hbm():
            acc = jnp.dot(a_ref[...], h_scr[...],
                          preferred_element_type=jnp.float32)
            out_ref[...] = acc * norm + b_ref[...]

        @pl.when(i >= NI - CI)
        def _from_cache():
            a = cache_scr[pl.ds((i - (NI - CI)) * BM, BM), :]
            acc = jnp.dot(a, hb_scr[...], preferred_element_type=jnp.float32)
            out_ref[...] = acc * norm + b_ref[...]


def _a_index(p, i):
    return (jnp.where(p == 0, i, jnp.minimum(i, NI - CI - 1)), 0)


def kernel(Adjacency, Features, W, b):
    assert Adjacency.shape == (N, N)
    assert Features.shape == (N, D)

    out = pl.pallas_call(
        _gcn_kernel,
        grid=(2, NI),
        in_specs=[
            pl.BlockSpec((BM, N), _a_index),
            pl.BlockSpec((BM, D), lambda p, i: (i, 0)),
            pl.BlockSpec((D, D), lambda p, i: (0, 0)),
            pl.BlockSpec((1, D), lambda p, i: (0, 0)),
        ],
        out_specs=pl.BlockSpec((BM, D), lambda p, i: (p * i, 0)),
        out_shape=jax.ShapeDtypeStruct((N, D), jnp.float32),
        scratch_shapes=[
            pltpu.VMEM((N, D), jnp.float32),
            pltpu.VMEM((N, D), jnp.bfloat16),
            pltpu.VMEM((N, 1), jnp.float32),
            pltpu.VMEM((CI * BM, N), jnp.bfloat16),
        ],
        compiler_params=pltpu.CompilerParams(
            dimension_semantics=("arbitrary", "arbitrary")),
    )(Adjacency, Features, W, b.reshape(1, D))
    return out


# speedup vs baseline: 1.3421x; 1.0051x over previous
import jax
import jax.numpy as jnp
from jax.experimental import pallas as pl
from jax.experimental.pallas import tpu as pltpu

N = 10000
D = 128
BM = 200
NI = N // BM
CI = 7


def _gcn_kernel(a_ref, f_ref, w_ref, b_ref, out_ref,
                hb_scr, norm_scr, cache_scr):
    phase = pl.program_id(0)
    i = pl.program_id(1)

    @pl.when(phase == 0)
    def _deg_h():
        a = a_ref[...]
        deg = jnp.sum(a, axis=1, keepdims=True)
        norm = jnp.where(deg > 0.0, jax.lax.rsqrt(deg), 0.0)
        norm_scr[pl.ds(i * BM, BM), :] = norm
        fw = jax.lax.dot_general(
            f_ref[...], w_ref[...],
            dimension_numbers=(((1,), (1,)), ((), ())),
            preferred_element_type=jnp.float32,
        )
        hb_scr[pl.ds(i * BM, BM), :] = (fw * norm).astype(jnp.bfloat16)

        @pl.when(i >= NI - CI)
        def _stash():
            cache_scr[pl.ds((i - (NI - CI)) * BM, BM), :] = (
                a.astype(jnp.bfloat16))

    @pl.when(phase == 1)
    def _spmm():
        norm = norm_scr[pl.ds(i * BM, BM), :]

        @pl.when(i < NI - CI)
        def _from_hbm():
            a = a_ref[...].astype(jnp.bfloat16)
            acc = jnp.dot(a, hb_scr[...], preferred_element_type=jnp.float32)
            out_ref[...] = acc * norm + b_ref[...]

        @pl.when(i >= NI - CI)
        def _from_cache():
            a = cache_scr[pl.ds((i - (NI - CI)) * BM, BM), :]
            acc = jnp.dot(a, hb_scr[...], preferred_element_type=jnp.float32)
            out_ref[...] = acc * norm + b_ref[...]


def _a_index(p, i):
    return (jnp.where(p == 0, i, jnp.minimum(i, NI - CI - 1)), 0)


def kernel(Adjacency, Features, W, b):
    assert Adjacency.shape == (N, N)
    assert Features.shape == (N, D)

    out = pl.pallas_call(
        _gcn_kernel,
        grid=(2, NI),
        in_specs=[
            pl.BlockSpec((BM, N), _a_index),
            pl.BlockSpec((BM, D), lambda p, i: (i, 0)),
            pl.BlockSpec((D, D), lambda p, i: (0, 0)),
            pl.BlockSpec((1, D), lambda p, i: (0, 0)),
        ],
        out_specs=pl.BlockSpec((BM, D), lambda p, i: (p * i, 0)),
        out_shape=jax.ShapeDtypeStruct((N, D), jnp.float32),
        scratch_shapes=[
            pltpu.VMEM((N, D), jnp.bfloat16),
            pltpu.VMEM((N, 1), jnp.float32),
            pltpu.VMEM((CI * BM, N), jnp.bfloat16),
        ],
        compiler_params=pltpu.CompilerParams(
            dimension_semantics=("arbitrary", "arbitrary")),
    )(Adjacency, Features, W, b.reshape(1, D))
    return out
